# baseline (speedup 1.0000x reference)
import functools

import jax
import jax.numpy as jnp
from jax import lax
from jax.experimental import pallas as pl
from jax.experimental.pallas import tpu as pltpu
from jax.experimental.pallas import tpu_sc as plsc

_CHUNK = 128
_DP = 128


@functools.lru_cache(maxsize=None)
def _make_gather_kernel(B, D, NC, NS):
    NW = NC * NS
    b_per_w = B // NW
    n_chunks = b_per_w // _CHUNK
    mesh = plsc.VectorSubcoreMesh(core_axis_name="c", subcore_axis_name="s")

    @functools.partial(
        pl.kernel,
        mesh=mesh,
        out_type=jax.ShapeDtypeStruct((B, D), jnp.float32),
        scratch_types=[
            pltpu.VMEM((n_chunks, _CHUNK), jnp.int32),
            pltpu.VMEM((b_per_w,), jnp.int32),
            pltpu.VMEM((b_per_w, _DP), jnp.float32),
            pltpu.VMEM((_CHUNK, D), jnp.float32),
            pltpu.SemaphoreType.DMA,
        ],
    )
    def gather_scale(idx_hbm, tab_hbm, out_hbm,
                     idx_v, par_v, pairs_v, orow_v, sem):
        wid = lax.axis_index("s") * NC + lax.axis_index("c")
        base = wid * b_per_w
        for j in range(n_chunks):
            pltpu.sync_copy(idx_hbm.at[pl.ds(base + j * _CHUNK, _CHUNK)],
                            idx_v.at[j])
        for j in range(n_chunks):
            for h in range(_CHUNK // 16):
                sl = pl.ds(h * 16, 16)
                v = idx_v[j, sl]
                par_v[pl.ds(j * _CHUNK + h * 16, 16)] = v & 1
                idx_v[j, sl] = v >> 1
        copies = [
            pltpu.async_copy(
                tab_hbm.at[idx_v.at[j]],
                pairs_v.at[pl.ds(j * _CHUNK, _CHUNK)], sem)
            for j in range(n_chunks)
        ]
        for j, c in enumerate(copies):
            c.wait()

            def group_body(g, carry):
                pv = par_v[pl.ds(j * _CHUNK + g * 16, 16)]
                for s in range(16):
                    r = g * 16 + s
                    off = pv[s] * D
                    for k in range(D // 16):
                        orow_v[r, pl.ds(k * 16, 16)] = (
                            pairs_v[j * _CHUNK + r,
                                    pl.ds(off + k * 16, 16)] * 0.25)
                return carry

            lax.fori_loop(0, _CHUNK // 16, group_body, 0)
            pltpu.sync_copy(
                orow_v, out_hbm.at[pl.ds(base + j * _CHUNK, _CHUNK)])

    return gather_scale


def kernel(users, items, user_table, item_table, edge_user, edge_item):
    B = users.shape[0]
    D = user_table.shape[1]
    info = plsc.get_sparse_core_info()
    fn = _make_gather_kernel(B, D, info.num_cores, info.num_subcores)
    utp = user_table.reshape(user_table.shape[0] // 2, _DP)
    itp = item_table.reshape(item_table.shape[0] // 2, _DP)
    return fn(users, utp), fn(items, itp)


# speedup vs baseline: 1.0000x; 1.0000x over previous
import functools

import jax
import jax.numpy as jnp
from jax import lax
from jax.experimental import pallas as pl
from jax.experimental.pallas import tpu as pltpu
from jax.experimental.pallas import tpu_sc as plsc

_CHUNK = 128
_DP = 128


@functools.lru_cache(maxsize=None)
def _make_gather_kernel(B, D, NC, NS):
    NW = NC * NS
    b_per_w = B // NW
    n_chunks = b_per_w // _CHUNK
    mesh = plsc.VectorSubcoreMesh(core_axis_name="c", subcore_axis_name="s")

    @functools.partial(
        pl.kernel,
        mesh=mesh,
        out_type=jax.ShapeDtypeStruct((B, D), jnp.float32),
        scratch_types=[
            pltpu.VMEM((n_chunks, _CHUNK), jnp.int32),
            pltpu.VMEM((b_per_w,), jnp.int32),
            pltpu.VMEM((b_per_w, _DP), jnp.float32),
            pltpu.VMEM((_CHUNK, D), jnp.float32),
            pltpu.SemaphoreType.DMA,
        ],
    )
    def gather_scale(idx_hbm, tab_hbm, out_hbm,
                     idx_v, par_v, pairs_v, orow_v, sem):
        wid = lax.axis_index("s") * NC + lax.axis_index("c")
        base = wid * b_per_w
        for j in range(n_chunks):
            pltpu.sync_copy(idx_hbm.at[pl.ds(base + j * _CHUNK, _CHUNK)],
                            idx_v.at[j])
        for j in range(n_chunks):
            for h in range(_CHUNK // 16):
                sl = pl.ds(h * 16, 16)
                v = idx_v[j, sl]
                par_v[pl.ds(j * _CHUNK + h * 16, 16)] = v & 1
                idx_v[j, sl] = v >> 1
        copies = [
            pltpu.async_copy(
                tab_hbm.at[idx_v.at[j]],
                pairs_v.at[pl.ds(j * _CHUNK, _CHUNK)], sem)
            for j in range(n_chunks)
        ]
        for j, c in enumerate(copies):
            c.wait()

            def group_body(g, carry):
                pv = par_v[pl.ds(j * _CHUNK + g * 16, 16)]
                for s in range(16):
                    r = g * 16 + s
                    off = pv[s] * D
                    for k in range(D // 16):
                        orow_v[r, pl.ds(k * 16, 16)] = (
                            pairs_v[j * _CHUNK + r,
                                    pl.ds(off + k * 16, 16)] * 0.25)
                return carry

            lax.fori_loop(0, _CHUNK // 16, group_body, 0)
            pltpu.sync_copy(
                orow_v, out_hbm.at[pl.ds(base + j * _CHUNK, _CHUNK)])

    return gather_scale


def kernel(users, items, user_table, item_table, edge_user, edge_item):
    B = users.shape[0]
    D = user_table.shape[1]
    info = plsc.get_sparse_core_info()
    fn = _make_gather_kernel(B, D, info.num_cores, info.num_subcores)
    utp = lax.optimization_barrier(
        user_table.reshape(user_table.shape[0] // 2, _DP))
    itp = lax.optimization_barrier(
        item_table.reshape(item_table.shape[0] // 2, _DP))
    return fn(users, utp), fn(items, itp)
